# baseline (speedup 1.0000x reference)
import functools

import jax
import jax.numpy as jnp
from jax import lax
from jax.experimental import pallas as pl
from jax.experimental.pallas import tpu as pltpu
from jax.experimental.pallas import tpu_sc as plsc

HID = 1024
F = 32
SEQ = F * F
B = 16
NT = B * SEQ
EPS = 1e-12

NC = 2
NS = 16
NW = NC * NS

P = 4
CH = NT // P
RPW = CH // NW
K = 32
NGC = RPW // K

ROWS_BLK = 256
GRID_P = CH // ROWS_BLK
A1B = ROWS_BLK // F


def _sc_gather_chunk(table, ids3d):
    mesh = plsc.VectorSubcoreMesh(core_axis_name="c", subcore_axis_name="s")

    @functools.partial(
        pl.kernel,
        mesh=mesh,
        out_type=jax.ShapeDtypeStruct((CH, HID // 2), jnp.int32),
        scratch_types=[
            pltpu.VMEM((NGC, K), jnp.int32),
            pltpu.VMEM((K, HID), jnp.float32),
            pltpu.VMEM((K, HID), jnp.float32),
            pltpu.VMEM((K, HID // 2), jnp.int32),
            pltpu.VMEM((K, HID // 2), jnp.int32),
            pltpu.SemaphoreType.DMA,
            pltpu.SemaphoreType.DMA,
            pltpu.SemaphoreType.DMA,
            pltpu.SemaphoreType.DMA,
        ],
    )
    def body(table_hbm, ids_hbm, out_hbm, idx_v, fbuf0, fbuf1, bbuf0, bbuf1,
             g0, g1, w0, w1):
        wid = lax.axis_index("s") * NC + lax.axis_index("c")
        base = wid * RPW
        pltpu.sync_copy(ids_hbm.at[wid], idx_v)
        fbufs = [fbuf0, fbuf1]
        bbufs = [bbuf0, bbuf1]
        gsem = [g0, g1]
        wsem = [w0, w1]
        gcp = [None, None]
        wcp = [None, None]

        iota = lax.iota(jnp.int32, 16)

        def _rnd(u):
            lsb = lax.shift_right_logical(u, 16) & 1
            return lax.shift_right_logical(u + 0x7FFF + lsb, 16)

        def pack_chunk(fbuf, bbuf):
            def pack_row(r, _):
                for w in range(HID // 32):
                    a = fbuf[r, pl.ds(w * 32, 16)]
                    b = fbuf[r, pl.ds(w * 32 + 16, 16)]
                    re = _rnd(lax.bitcast_convert_type(a, jnp.int32))
                    ro = _rnd(lax.bitcast_convert_type(b, jnp.int32))
                    bbuf[r, pl.ds(w * 16, 16)] = re | lax.shift_left(ro, 16)
                return 0
            lax.fori_loop(0, K, pack_row, 0)

        gcp[0] = pltpu.async_copy(table_hbm.at[idx_v.at[0]], fbufs[0], gsem[0])
        for g in range(NGC):
            bi = g & 1
            ni = (g + 1) & 1
            if g + 1 < NGC:
                gcp[ni] = pltpu.async_copy(
                    table_hbm.at[idx_v.at[g + 1]], fbufs[ni], gsem[ni]
                )
            gcp[bi].wait()
            if wcp[bi] is not None:
                wcp[bi].wait()
            pack_chunk(fbufs[bi], bbufs[bi])
            wcp[bi] = pltpu.async_copy(
                bbufs[bi], out_hbm.at[pl.ds(base + g * K, K)], wsem[bi]
            )
        wcp[0].wait()
        wcp[1].wait()

    return body(table, ids3d)


def _tc_chunk(p, carry, emb, ax1, ax2, ttpad, dtt, gamma, beta):
    def body(*refs):
        if carry is None:
            emb_ref, ax1_ref, ax2_ref, tt_ref, dtt_ref, g_ref, b_ref, o_ref = refs
        else:
            _, emb_ref, ax1_ref, ax2_ref, tt_ref, dtt_ref, g_ref, b_ref, o_ref = refs
        u = emb_ref[...]
        lo = lax.bitcast_convert_type(lax.shift_left(u, 16), jnp.float32)
        hi = lax.bitcast_convert_type(u & jnp.int32(-65536), jnp.float32)
        x = jnp.concatenate(
            [lo.reshape(ROWS_BLK, F, 1, 16), hi.reshape(ROWS_BLK, F, 1, 16)],
            axis=2,
        ).reshape(A1B, F, HID)
        x = x + ax1_ref[...][:, None, :] + ax2_ref[...][None, :, :]
        ttf = jnp.sum(tt_ref[...], axis=1, keepdims=True)
        x = x + (ttf * dtt_ref[...]).reshape(A1B, F, HID)
        mean = jnp.mean(x, axis=2, keepdims=True)
        var = jnp.mean(x * x, axis=2, keepdims=True) - mean * mean
        inv = lax.rsqrt(var + EPS)
        y = (x - mean) * inv * g_ref[...][None, :, :] + b_ref[...][None, :, :]
        o_ref[...] = y.reshape(ROWS_BLK, HID)

    specs = [
        pl.BlockSpec((ROWS_BLK, HID // 2), lambda t: (t, 0)),
        pl.BlockSpec((A1B, HID), lambda t: (t % (F // A1B), 0)),
        pl.BlockSpec((F, HID), lambda t: (0, 0)),
        pl.BlockSpec((ROWS_BLK, 8), lambda t: (t, 0)),
        pl.BlockSpec((1, HID), lambda t: (0, 0)),
        pl.BlockSpec((1, HID), lambda t: (0, 0)),
        pl.BlockSpec((1, HID), lambda t: (0, 0)),
    ]
    args = (emb, ax1, ax2, ttpad, dtt, gamma, beta)
    aliases = {}
    if carry is not None:
        specs = [pl.BlockSpec(memory_space=pl.ANY)] + specs
        args = (carry,) + args
        aliases = {0: 0}
    return pl.pallas_call(
        body,
        grid=(GRID_P,),
        in_specs=specs,
        out_specs=pl.BlockSpec(
            (ROWS_BLK, HID), lambda t, p=p: (p * GRID_P + t, 0)
        ),
        out_shape=jax.ShapeDtypeStruct((NT, HID), jnp.float32),
        input_output_aliases=aliases,
    )(*args)


def kernel(input_ids, token_type_ids, word_table, ax1, ax2, tt_table, gamma,
           beta):
    ids4d = input_ids.astype(jnp.int32).reshape(P, NW, NGC, K)
    ttf = token_type_ids.astype(jnp.float32).reshape(NT, 1)
    ttpad = jnp.pad(ttf, ((0, 0), (0, 7)))
    ax1p = ax1 + tt_table[0:1]
    dtt = tt_table[1:2] - tt_table[0:1]
    g2 = gamma.reshape(1, HID)
    b2 = beta.reshape(1, HID)

    embs = [_sc_gather_chunk(word_table, ids4d[p]) for p in range(P)]
    out = None
    for p in range(P):
        out = _tc_chunk(p, out, embs[p], ax1p, ax2,
                        ttpad[p * CH:(p + 1) * CH], dtt, g2, b2)
    return out.reshape(B, SEQ, HID)


# speedup vs baseline: 4.7887x; 4.7887x over previous
import functools

import jax
import jax.numpy as jnp
from jax import lax
from jax.experimental import pallas as pl
from jax.experimental.pallas import tpu as pltpu
from jax.experimental.pallas import tpu_sc as plsc

HID = 1024
F = 32
SEQ = F * F
B = 16
NT = B * SEQ
EPS = 1e-12

NC = 2
NS = 16
NW = NC * NS

P = 4
CH = NT // P
RPW = CH // NW
K = 32
NGC = RPW // K

ROWS_BLK = 256
GRID_P = CH // ROWS_BLK
A1B = ROWS_BLK // F


def _sc_gather_chunk(table, ids3d, tts3d):
    mesh = plsc.VectorSubcoreMesh(core_axis_name="c", subcore_axis_name="s")

    @functools.partial(
        pl.kernel,
        mesh=mesh,
        out_type=(
            jax.ShapeDtypeStruct((CH, HID), jnp.float32),
            jax.ShapeDtypeStruct((CH, 16), jnp.int32),
        ),
        scratch_types=[
            pltpu.VMEM((NGC, K), jnp.int32),
            pltpu.VMEM((RPW // 16, 16), jnp.int32),
            pltpu.VMEM((RPW, 16), jnp.int32),
            pltpu.VMEM((K, HID), jnp.float32),
            pltpu.VMEM((K, HID), jnp.float32),
            pltpu.SemaphoreType.DMA,
            pltpu.SemaphoreType.DMA,
            pltpu.SemaphoreType.DMA,
            pltpu.SemaphoreType.DMA,
            pltpu.SemaphoreType.DMA,
        ],
    )
    def body(table_hbm, ids_hbm, tts_hbm, out_hbm, tt_hbm, idx_v, tt_v, tbuf,
             buf0, buf1, g0, g1, w0, w1, t0):
        wid = lax.axis_index("s") * NC + lax.axis_index("c")
        base = wid * RPW
        pltpu.sync_copy(ids_hbm.at[wid], idx_v)
        pltpu.sync_copy(tts_hbm.at[wid], tt_v)
        def tt_fill(q, _):
            v = tt_v[q]
            for j in range(16):
                tbuf[q * 16 + j] = jnp.full((16,), v[j], jnp.int32)
            return 0
        lax.fori_loop(0, RPW // 16, tt_fill, 0)
        ttcp = pltpu.async_copy(tbuf, tt_hbm.at[pl.ds(base, RPW)], t0)
        bufs = [buf0, buf1]
        gsem = [g0, g1]
        wsem = [w0, w1]
        gcp = [None, None]
        wcp = [None, None]
        gcp[0] = pltpu.async_copy(table_hbm.at[idx_v.at[0]], bufs[0], gsem[0])
        for g in range(NGC):
            bi = g & 1
            ni = (g + 1) & 1
            if g + 1 < NGC:
                if wcp[ni] is not None:
                    wcp[ni].wait()
                gcp[ni] = pltpu.async_copy(
                    table_hbm.at[idx_v.at[g + 1]], bufs[ni], gsem[ni]
                )
            gcp[bi].wait()
            wcp[bi] = pltpu.async_copy(
                bufs[bi], out_hbm.at[pl.ds(base + g * K, K)], wsem[bi]
            )
        wcp[0].wait()
        wcp[1].wait()
        ttcp.wait()

    return body(table, ids3d, tts3d)


def _tc_chunk(p, carry, emb, ax1, ax2, tt16, dtt, gamma, beta):
    def body(*refs):
        if carry is None:
            emb_ref, ax1_ref, ax2_ref, tt_ref, dtt_ref, g_ref, b_ref, o_ref = refs
        else:
            _, emb_ref, ax1_ref, ax2_ref, tt_ref, dtt_ref, g_ref, b_ref, o_ref = refs
        x = emb_ref[...].reshape(A1B, F, HID)
        x = x + ax1_ref[...][:, None, :] + ax2_ref[...][None, :, :]
        ttf = jnp.sum(tt_ref[...].astype(jnp.float32), axis=1,
                      keepdims=True) * (1.0 / 16.0)
        x = x + (ttf * dtt_ref[...]).reshape(A1B, F, HID)
        mean = jnp.mean(x, axis=2, keepdims=True)
        var = jnp.mean(x * x, axis=2, keepdims=True) - mean * mean
        inv = lax.rsqrt(var + EPS)
        y = (x - mean) * inv * g_ref[...][None, :, :] + b_ref[...][None, :, :]
        o_ref[...] = y.reshape(ROWS_BLK, HID)

    specs = [
        pl.BlockSpec((ROWS_BLK, HID), lambda t: (t, 0)),
        pl.BlockSpec((A1B, HID), lambda t: (t % (F // A1B), 0)),
        pl.BlockSpec((F, HID), lambda t: (0, 0)),
        pl.BlockSpec((ROWS_BLK, 16), lambda t: (t, 0)),
        pl.BlockSpec((1, HID), lambda t: (0, 0)),
        pl.BlockSpec((1, HID), lambda t: (0, 0)),
        pl.BlockSpec((1, HID), lambda t: (0, 0)),
    ]
    args = (emb, ax1, ax2, tt16, dtt, gamma, beta)
    aliases = {}
    if carry is not None:
        specs = [pl.BlockSpec(memory_space=pl.ANY)] + specs
        args = (carry,) + args
        aliases = {0: 0}
    return pl.pallas_call(
        body,
        grid=(GRID_P,),
        in_specs=specs,
        out_specs=pl.BlockSpec(
            (ROWS_BLK, HID), lambda t, p=p: (p * GRID_P + t, 0)
        ),
        out_shape=jax.ShapeDtypeStruct((NT, HID), jnp.float32),
        input_output_aliases=aliases,
    )(*args)


def kernel(input_ids, token_type_ids, word_table, ax1, ax2, tt_table, gamma,
           beta):
    ids4d = input_ids.astype(jnp.int32).reshape(P, NW, NGC, K)
    tts4d = token_type_ids.astype(jnp.int32).reshape(P, NW, RPW // 16, 16)
    ax1p = ax1 + tt_table[0:1]
    dtt = tt_table[1:2] - tt_table[0:1]
    g2 = gamma.reshape(1, HID)
    b2 = beta.reshape(1, HID)

    scs = [_sc_gather_chunk(word_table, ids4d[p], tts4d[p]) for p in range(P)]
    out = None
    for p in range(P):
        emb_p, tt_p = scs[p]
        out = _tc_chunk(p, out, emb_p, ax1p, ax2, tt_p, dtt, g2, b2)
    return out.reshape(B, SEQ, HID)


# speedup vs baseline: 5.0125x; 1.0467x over previous
import functools

import jax
import jax.numpy as jnp
from jax import lax
from jax.experimental import pallas as pl
from jax.experimental.pallas import tpu as pltpu
from jax.experimental.pallas import tpu_sc as plsc

HID = 1024
F = 32
SEQ = F * F
B = 16
NT = B * SEQ
EPS = 1e-12

NC = 2
NS = 16
NW = NC * NS

P = 4
CH = NT // P
RPW = CH // NW
K = 32
NGC = RPW // K

ROWS_BLK = 512
GRID_P = CH // ROWS_BLK
A1B = ROWS_BLK // F


def _sc_gather_chunk(table, ids3d, tts3d):
    mesh = plsc.VectorSubcoreMesh(core_axis_name="c", subcore_axis_name="s")

    @functools.partial(
        pl.kernel,
        mesh=mesh,
        out_type=(
            jax.ShapeDtypeStruct((CH, HID), jnp.float32),
            jax.ShapeDtypeStruct((CH, 16), jnp.int32),
        ),
        scratch_types=[
            pltpu.VMEM((NGC, K), jnp.int32),
            pltpu.VMEM((RPW // 16, 16), jnp.int32),
            pltpu.VMEM((RPW, 16), jnp.int32),
            pltpu.VMEM((K, HID), jnp.float32),
            pltpu.VMEM((K, HID), jnp.float32),
            pltpu.SemaphoreType.DMA,
            pltpu.SemaphoreType.DMA,
            pltpu.SemaphoreType.DMA,
            pltpu.SemaphoreType.DMA,
            pltpu.SemaphoreType.DMA,
        ],
    )
    def body(table_hbm, ids_hbm, tts_hbm, out_hbm, tt_hbm, idx_v, tt_v, tbuf,
             buf0, buf1, g0, g1, w0, w1, t0):
        wid = lax.axis_index("s") * NC + lax.axis_index("c")
        base = wid * RPW
        pltpu.sync_copy(ids_hbm.at[wid], idx_v)
        pltpu.sync_copy(tts_hbm.at[wid], tt_v)
        def tt_fill(q, _):
            v = tt_v[q]
            for j in range(16):
                tbuf[q * 16 + j] = jnp.full((16,), v[j], jnp.int32)
            return 0
        lax.fori_loop(0, RPW // 16, tt_fill, 0)
        ttcp = pltpu.async_copy(tbuf, tt_hbm.at[pl.ds(base, RPW)], t0)
        bufs = [buf0, buf1]
        gsem = [g0, g1]
        wsem = [w0, w1]
        gcp = [None, None]
        wcp = [None, None]
        gcp[0] = pltpu.async_copy(table_hbm.at[idx_v.at[0]], bufs[0], gsem[0])
        for g in range(NGC):
            bi = g & 1
            ni = (g + 1) & 1
            if g + 1 < NGC:
                if wcp[ni] is not None:
                    wcp[ni].wait()
                gcp[ni] = pltpu.async_copy(
                    table_hbm.at[idx_v.at[g + 1]], bufs[ni], gsem[ni]
                )
            gcp[bi].wait()
            wcp[bi] = pltpu.async_copy(
                bufs[bi], out_hbm.at[pl.ds(base + g * K, K)], wsem[bi]
            )
        wcp[0].wait()
        wcp[1].wait()
        ttcp.wait()

    return body(table, ids3d, tts3d)


def _tc_chunk(p, carry, emb, ax1, ax2, tt16, dtt, gamma, beta):
    def body(*refs):
        if carry is None:
            emb_ref, ax1_ref, ax2_ref, tt_ref, dtt_ref, g_ref, b_ref, o_ref = refs
        else:
            _, emb_ref, ax1_ref, ax2_ref, tt_ref, dtt_ref, g_ref, b_ref, o_ref = refs
        x = emb_ref[...].reshape(A1B, F, HID)
        x = x + ax1_ref[...][:, None, :] + ax2_ref[...][None, :, :]
        ttf = jnp.sum(tt_ref[...].astype(jnp.float32), axis=1,
                      keepdims=True) * (1.0 / 16.0)
        x = x + (ttf * dtt_ref[...]).reshape(A1B, F, HID)
        mean = jnp.mean(x, axis=2, keepdims=True)
        var = jnp.mean(x * x, axis=2, keepdims=True) - mean * mean
        inv = lax.rsqrt(var + EPS)
        y = (x - mean) * inv * g_ref[...][None, :, :] + b_ref[...][None, :, :]
        o_ref[...] = y.reshape(ROWS_BLK, HID)

    specs = [
        pl.BlockSpec((ROWS_BLK, HID), lambda t: (t, 0)),
        pl.BlockSpec((A1B, HID), lambda t: (t % (F // A1B), 0)),
        pl.BlockSpec((F, HID), lambda t: (0, 0)),
        pl.BlockSpec((ROWS_BLK, 16), lambda t: (t, 0)),
        pl.BlockSpec((1, HID), lambda t: (0, 0)),
        pl.BlockSpec((1, HID), lambda t: (0, 0)),
        pl.BlockSpec((1, HID), lambda t: (0, 0)),
    ]
    args = (emb, ax1, ax2, tt16, dtt, gamma, beta)
    aliases = {}
    if carry is not None:
        specs = [pl.BlockSpec(memory_space=pl.ANY)] + specs
        args = (carry,) + args
        aliases = {0: 0}
    return pl.pallas_call(
        body,
        grid=(GRID_P,),
        in_specs=specs,
        out_specs=pl.BlockSpec(
            (ROWS_BLK, HID), lambda t, p=p: (p * GRID_P + t, 0)
        ),
        out_shape=jax.ShapeDtypeStruct((NT, HID), jnp.float32),
        input_output_aliases=aliases,
    )(*args)


def kernel(input_ids, token_type_ids, word_table, ax1, ax2, tt_table, gamma,
           beta):
    ids4d = input_ids.astype(jnp.int32).reshape(P, NW, NGC, K)
    tts4d = token_type_ids.astype(jnp.int32).reshape(P, NW, RPW // 16, 16)
    ax1p = ax1 + tt_table[0:1]
    dtt = tt_table[1:2] - tt_table[0:1]
    g2 = gamma.reshape(1, HID)
    b2 = beta.reshape(1, HID)

    scs = [_sc_gather_chunk(word_table, ids4d[p], tts4d[p]) for p in range(P)]
    out = None
    for p in range(P):
        emb_p, tt_p = scs[p]
        out = _tc_chunk(p, out, emb_p, ax1p, ax2, tt_p, dtt, g2, b2)
    return out.reshape(B, SEQ, HID)


# speedup vs baseline: 5.0894x; 1.0154x over previous
import functools

import jax
import jax.numpy as jnp
from jax import lax
from jax.experimental import pallas as pl
from jax.experimental.pallas import tpu as pltpu
from jax.experimental.pallas import tpu_sc as plsc

HID = 1024
F = 32
SEQ = F * F
B = 16
NT = B * SEQ
EPS = 1e-12

NC = 2
NS = 16
NW = NC * NS

P = 4
CH = NT // P
RPW = CH // NW
K = 32
NGC = RPW // K

ROWS_BLK = 1024
GRID_P = CH // ROWS_BLK
A1B = ROWS_BLK // F


def _sc_gather_chunk(table, ids3d, tts3d):
    mesh = plsc.VectorSubcoreMesh(core_axis_name="c", subcore_axis_name="s")

    @functools.partial(
        pl.kernel,
        mesh=mesh,
        out_type=(
            jax.ShapeDtypeStruct((CH, HID), jnp.float32),
            jax.ShapeDtypeStruct((CH, 16), jnp.int32),
        ),
        scratch_types=[
            pltpu.VMEM((NGC, K), jnp.int32),
            pltpu.VMEM((RPW // 16, 16), jnp.int32),
            pltpu.VMEM((RPW, 16), jnp.int32),
            pltpu.VMEM((K, HID), jnp.float32),
            pltpu.VMEM((K, HID), jnp.float32),
            pltpu.SemaphoreType.DMA,
            pltpu.SemaphoreType.DMA,
            pltpu.SemaphoreType.DMA,
            pltpu.SemaphoreType.DMA,
            pltpu.SemaphoreType.DMA,
        ],
    )
    def body(table_hbm, ids_hbm, tts_hbm, out_hbm, tt_hbm, idx_v, tt_v, tbuf,
             buf0, buf1, g0, g1, w0, w1, t0):
        wid = lax.axis_index("s") * NC + lax.axis_index("c")
        base = wid * RPW
        pltpu.sync_copy(ids_hbm.at[wid], idx_v)
        pltpu.sync_copy(tts_hbm.at[wid], tt_v)
        def tt_fill(q, _):
            v = tt_v[q]
            for j in range(16):
                tbuf[q * 16 + j] = jnp.full((16,), v[j], jnp.int32)
            return 0
        lax.fori_loop(0, RPW // 16, tt_fill, 0)
        ttcp = pltpu.async_copy(tbuf, tt_hbm.at[pl.ds(base, RPW)], t0)
        bufs = [buf0, buf1]
        gsem = [g0, g1]
        wsem = [w0, w1]
        gcp = [None, None]
        wcp = [None, None]
        gcp[0] = pltpu.async_copy(table_hbm.at[idx_v.at[0]], bufs[0], gsem[0])
        for g in range(NGC):
            bi = g & 1
            ni = (g + 1) & 1
            if g + 1 < NGC:
                if wcp[ni] is not None:
                    wcp[ni].wait()
                gcp[ni] = pltpu.async_copy(
                    table_hbm.at[idx_v.at[g + 1]], bufs[ni], gsem[ni]
                )
            gcp[bi].wait()
            wcp[bi] = pltpu.async_copy(
                bufs[bi], out_hbm.at[pl.ds(base + g * K, K)], wsem[bi]
            )
        wcp[0].wait()
        wcp[1].wait()
        ttcp.wait()

    return body(table, ids3d, tts3d)


def _tc_chunk(p, carry, emb, ax1, ax2, tt16, dtt, gamma, beta):
    def body(*refs):
        if carry is None:
            emb_ref, ax1_ref, ax2_ref, tt_ref, dtt_ref, g_ref, b_ref, o_ref = refs
        else:
            _, emb_ref, ax1_ref, ax2_ref, tt_ref, dtt_ref, g_ref, b_ref, o_ref = refs
        x = emb_ref[...].reshape(A1B, F, HID)
        x = x + ax1_ref[...][:, None, :] + ax2_ref[...][None, :, :]
        ttf = jnp.sum(tt_ref[...].astype(jnp.float32), axis=1,
                      keepdims=True) * (1.0 / 16.0)
        x = x + (ttf * dtt_ref[...]).reshape(A1B, F, HID)
        mean = jnp.mean(x, axis=2, keepdims=True)
        var = jnp.mean(x * x, axis=2, keepdims=True) - mean * mean
        inv = lax.rsqrt(var + EPS)
        y = (x - mean) * inv * g_ref[...][None, :, :] + b_ref[...][None, :, :]
        o_ref[...] = y.reshape(ROWS_BLK, HID)

    specs = [
        pl.BlockSpec((ROWS_BLK, HID), lambda t: (t, 0)),
        pl.BlockSpec((A1B, HID), lambda t: (t % (F // A1B), 0)),
        pl.BlockSpec((F, HID), lambda t: (0, 0)),
        pl.BlockSpec((ROWS_BLK, 16), lambda t: (t, 0)),
        pl.BlockSpec((1, HID), lambda t: (0, 0)),
        pl.BlockSpec((1, HID), lambda t: (0, 0)),
        pl.BlockSpec((1, HID), lambda t: (0, 0)),
    ]
    args = (emb, ax1, ax2, tt16, dtt, gamma, beta)
    aliases = {}
    if carry is not None:
        specs = [pl.BlockSpec(memory_space=pl.ANY)] + specs
        args = (carry,) + args
        aliases = {0: 0}
    return pl.pallas_call(
        body,
        grid=(GRID_P,),
        in_specs=specs,
        out_specs=pl.BlockSpec(
            (ROWS_BLK, HID), lambda t, p=p: (p * GRID_P + t, 0)
        ),
        out_shape=jax.ShapeDtypeStruct((NT, HID), jnp.float32),
        input_output_aliases=aliases,
    )(*args)


def kernel(input_ids, token_type_ids, word_table, ax1, ax2, tt_table, gamma,
           beta):
    ids4d = input_ids.astype(jnp.int32).reshape(P, NW, NGC, K)
    tts4d = token_type_ids.astype(jnp.int32).reshape(P, NW, RPW // 16, 16)
    ax1p = ax1 + tt_table[0:1]
    dtt = tt_table[1:2] - tt_table[0:1]
    g2 = gamma.reshape(1, HID)
    b2 = beta.reshape(1, HID)

    scs = [_sc_gather_chunk(word_table, ids4d[p], tts4d[p]) for p in range(P)]
    out = None
    for p in range(P):
        emb_p, tt_p = scs[p]
        out = _tc_chunk(p, out, emb_p, ax1p, ax2, tt_p, dtt, g2, b2)
    return out.reshape(B, SEQ, HID)
